# Initial kernel scaffold; baseline (speedup 1.0000x reference)
import jax
import jax.numpy as jnp
from jax.experimental import pallas as pl


def kernel(token_id, emb_norm):
    raise NotImplementedError("write your pallas kernel here")


# speedup vs baseline: 15.3487x; 15.3487x over previous
import functools

import jax
import jax.numpy as jnp
from jax import lax
from jax.experimental import pallas as pl
from jax.experimental.pallas import tpu as pltpu
from jax.experimental.pallas import tpu_sc as plsc

_K = 16
_T = 1024


def _gather_sc(token_id, emb_norm):
    V, D = emb_norm.shape
    B = token_id.shape[0]
    info = plsc.get_sparse_core_info()
    nw = info.num_cores * info.num_subcores
    b_per_w = B // nw
    mesh = plsc.VectorSubcoreMesh(core_axis_name="c", subcore_axis_name="s")

    @functools.partial(
        pl.kernel,
        mesh=mesh,
        out_type=jax.ShapeDtypeStruct((B, D), jnp.float32),
        scratch_types=[
            pltpu.VMEM((b_per_w,), jnp.int32),
            pltpu.VMEM((b_per_w, D), jnp.float32),
            pltpu.SemaphoreType.DMA,
        ],
    )
    def gather(idx_hbm, table_hbm, out_hbm, idx_v, rows_v, sem):
        wid = lax.axis_index("s") * info.num_cores + lax.axis_index("c")
        base = wid * b_per_w
        pltpu.sync_copy(idx_hbm.at[pl.ds(base, b_per_w)], idx_v)
        pltpu.async_copy(table_hbm.at[idx_v], rows_v, sem).wait()
        pltpu.sync_copy(rows_v, out_hbm.at[pl.ds(base, b_per_w)])

    return gather(token_id, emb_norm)


def _topk_body(q_ref, e_ref, vals_ref, idx_ref, qn_ref):
    B = q_ref.shape[0]
    T = e_ref.shape[0]

    @pl.when(pl.program_id(0) == 0)
    def _():
        q = q_ref[...]
        n = jnp.sqrt(jnp.sum(q * q, axis=1, keepdims=True))
        qn_ref[...] = q / jnp.maximum(n, 1e-12)

    qn = qn_ref[...]
    e = e_ref[...]
    s = lax.dot_general(qn, e, (((1,), (1,)), ((), ())),
                        preferred_element_type=jnp.float32)
    row = lax.broadcasted_iota(jnp.int32, (B, T), 0)
    neg = jnp.float32(-jnp.inf)
    for i in range(_K):
        m = jnp.max(s, axis=0)
        a = jnp.min(jnp.where(s == m[None, :], row, B), axis=0)
        vals_ref[i, :] = m
        idx_ref[i, :] = a.astype(jnp.int32)
        if i < _K - 1:
            s = jnp.where(row == a[None, :], neg, s)


def _topk_tc(q_raw, emb_norm):
    B, D = q_raw.shape
    V = emb_norm.shape[0]
    grid = pl.cdiv(V, _T)
    vals, idx = pl.pallas_call(
        _topk_body,
        grid=(grid,),
        in_specs=[
            pl.BlockSpec((B, D), lambda j: (0, 0)),
            pl.BlockSpec((_T, D), lambda j: (j, 0)),
        ],
        out_specs=[
            pl.BlockSpec((_K, _T), lambda j: (0, j)),
            pl.BlockSpec((_K, _T), lambda j: (0, j)),
        ],
        out_shape=[
            jax.ShapeDtypeStruct((_K, V), jnp.float32),
            jax.ShapeDtypeStruct((_K, V), jnp.int32),
        ],
        scratch_shapes=[pltpu.VMEM((B, D), jnp.float32)],
    )(q_raw, emb_norm)
    return idx, vals


def kernel(token_id, emb_norm):
    tok = token_id.reshape(-1).astype(jnp.int32)
    q = _gather_sc(tok, emb_norm)
    idx, vals = _topk_tc(q, emb_norm)
    return idx, vals


# speedup vs baseline: 29.1344x; 1.8982x over previous
import functools

import jax
import jax.numpy as jnp
from jax import lax
from jax.experimental import pallas as pl
from jax.experimental.pallas import tpu as pltpu
from jax.experimental.pallas import tpu_sc as plsc

_K = 16
_T = 1024
_NBLK = 16


def _oddeven_pairs(n):
    pairs = []
    p = 1
    while p < n:
        k = p
        while k >= 1:
            for j in range(k % p, n - k, 2 * k):
                for i in range(0, min(k, n - j - k)):
                    if (i + j) // (2 * p) == (i + j + k) // (2 * p):
                        pairs.append((i + j, i + j + k))
            k //= 2
        p *= 2
    return pairs


_SORT16 = _oddeven_pairs(16)
_BITONIC16 = [(i, i + d) for d in (8, 4, 2, 1) for i in range(16) if not (i & d)]


def _before(va, ia, vb, ib):
    return (va > vb) | ((va == vb) & (ia < ib))


def _gather_sc(token_id, emb_norm):
    V, D = emb_norm.shape
    B = token_id.shape[0]
    info = plsc.get_sparse_core_info()
    nw = info.num_cores * info.num_subcores
    b_per_w = B // nw
    mesh = plsc.VectorSubcoreMesh(core_axis_name="c", subcore_axis_name="s")

    @functools.partial(
        pl.kernel,
        mesh=mesh,
        out_type=jax.ShapeDtypeStruct((B, D), jnp.float32),
        scratch_types=[
            pltpu.VMEM((b_per_w,), jnp.int32),
            pltpu.VMEM((b_per_w, D), jnp.float32),
            pltpu.SemaphoreType.DMA,
        ],
    )
    def gather(idx_hbm, table_hbm, out_hbm, idx_v, rows_v, sem):
        wid = lax.axis_index("s") * info.num_cores + lax.axis_index("c")
        base = wid * b_per_w
        pltpu.sync_copy(idx_hbm.at[pl.ds(base, b_per_w)], idx_v)
        pltpu.async_copy(table_hbm.at[idx_v], rows_v, sem).wait()
        pltpu.sync_copy(rows_v, out_hbm.at[pl.ds(base, b_per_w)])

    return gather(token_id, emb_norm)


def _topk_body(q_ref, e_ref, vals_ref, idx_ref, qn_ref):
    B = q_ref.shape[0]
    T = e_ref.shape[0]

    @pl.when(pl.program_id(0) == 0)
    def _():
        q = q_ref[...]
        n = jnp.sqrt(jnp.sum(q * q, axis=1, keepdims=True))
        qn_ref[...] = q / jnp.maximum(n, 1e-12)

    qn = qn_ref[...]
    e = e_ref[...]
    s = lax.dot_general(qn, e, (((1,), (1,)), ((), ())),
                        preferred_element_type=jnp.float32)

    rows = B // _NBLK
    v = [s[i * rows:(i + 1) * rows] for i in range(_NBLK)]
    iota = lax.broadcasted_iota(jnp.int32, (rows, T), 0)
    ix = [iota + (i * rows) for i in range(_NBLK)]

    def ce(a, b):
        p = _before(v[a], ix[a], v[b], ix[b])
        hv = jnp.where(p, v[a], v[b])
        hi = jnp.where(p, ix[a], ix[b])
        lv = jnp.where(p, v[b], v[a])
        li = jnp.where(p, ix[b], ix[a])
        v[a], ix[a], v[b], ix[b] = hv, hi, lv, li

    for a, b in _SORT16:
        ce(a, b)

    half = rows // 2
    while half >= 1:
        nv, nix = [], []
        for i in range(_NBLK):
            j = _NBLK - 1 - i
            va, ia = v[i][:half], ix[i][:half]
            vb, ib = v[j][half:2 * half], ix[j][half:2 * half]
            p = _before(va, ia, vb, ib)
            nv.append(jnp.where(p, va, vb))
            nix.append(jnp.where(p, ia, ib))
        v, ix = nv, nix
        for a, b in _BITONIC16:
            ce(a, b)
        half //= 2

    for i in range(_K):
        vals_ref[i, :] = v[i][0]
        idx_ref[i, :] = ix[i][0]


def _topk_tc(q_raw, emb_norm):
    B, D = q_raw.shape
    V = emb_norm.shape[0]
    grid = pl.cdiv(V, _T)
    vals, idx = pl.pallas_call(
        _topk_body,
        grid=(grid,),
        in_specs=[
            pl.BlockSpec((B, D), lambda j: (0, 0)),
            pl.BlockSpec((_T, D), lambda j: (j, 0)),
        ],
        out_specs=[
            pl.BlockSpec((_K, _T), lambda j: (0, j)),
            pl.BlockSpec((_K, _T), lambda j: (0, j)),
        ],
        out_shape=[
            jax.ShapeDtypeStruct((_K, V), jnp.float32),
            jax.ShapeDtypeStruct((_K, V), jnp.int32),
        ],
        scratch_shapes=[pltpu.VMEM((B, D), jnp.float32)],
    )(q_raw, emb_norm)
    return idx, vals


def kernel(token_id, emb_norm):
    tok = token_id.reshape(-1).astype(jnp.int32)
    q = _gather_sc(tok, emb_norm)
    idx, vals = _topk_tc(q, emb_norm)
    return idx, vals


# speedup vs baseline: 39.0942x; 1.3419x over previous
import functools

import jax
import jax.numpy as jnp
import numpy as np
from jax import lax
from jax.experimental import pallas as pl
from jax.experimental.pallas import tpu as pltpu
from jax.experimental.pallas import tpu_sc as plsc

_K = 16
_T = 2048
_NBLK = 16
_ROWS = 64

_REV6 = np.array([int(f"{r:06b}"[::-1], 2) for r in range(_ROWS)])


def _oddeven_pairs(n):
    pairs = []
    p = 1
    while p < n:
        k = p
        while k >= 1:
            for j in range(k % p, n - k, 2 * k):
                for i in range(0, min(k, n - j - k)):
                    if (i + j) // (2 * p) == (i + j + k) // (2 * p):
                        pairs.append((i + j, i + j + k))
            k //= 2
        p *= 2
    return pairs


_SORT16 = _oddeven_pairs(16)
_BITONIC16 = [(i, i + d) for d in (8, 4, 2, 1) for i in range(16) if not (i & d)]

def _sort16_kinds():
    supports = [{s} for s in range(16)]
    kinds = []
    for a, b in _SORT16:
        kinds.append(max(supports[a]) < min(supports[b]))
        u = supports[a] | supports[b]
        supports[a], supports[b] = u, u
    return kinds


_SORT16_CHEAP = _sort16_kinds()


def _before(va, ia, vb, ib):
    return (va > vb) | ((va == vb) & (ia < ib))


def _gather_sc(token_id, emb_norm):
    V, D = emb_norm.shape
    B = token_id.shape[0]
    info = plsc.get_sparse_core_info()
    nw = info.num_cores * info.num_subcores
    b_per_w = B // nw
    mesh = plsc.VectorSubcoreMesh(core_axis_name="c", subcore_axis_name="s")

    @functools.partial(
        pl.kernel,
        mesh=mesh,
        out_type=jax.ShapeDtypeStruct((B, D), jnp.float32),
        scratch_types=[
            pltpu.VMEM((b_per_w,), jnp.int32),
            pltpu.VMEM((b_per_w, D), jnp.float32),
            pltpu.SemaphoreType.DMA,
        ],
    )
    def gather(idx_hbm, table_hbm, out_hbm, idx_v, rows_v, sem):
        wid = lax.axis_index("s") * info.num_cores + lax.axis_index("c")
        base = wid * b_per_w
        pltpu.sync_copy(idx_hbm.at[pl.ds(base, b_per_w)], idx_v)
        pltpu.async_copy(table_hbm.at[idx_v], rows_v, sem).wait()
        pltpu.sync_copy(rows_v, out_hbm.at[pl.ds(base, b_per_w)])

    return gather(token_id, emb_norm)


def _topk_body(q_ref, e_ref, vals_ref, idx_ref):
    B = q_ref.shape[0]
    T = e_ref.shape[0]
    qn = q_ref[...]
    e = e_ref[...]
    s = lax.dot_general(qn, e, (((1,), (1,)), ((), ())),
                        preferred_element_type=jnp.float32)

    rows = B // _NBLK
    v = [s[i * rows:(i + 1) * rows] for i in range(_NBLK)]
    r = lax.broadcasted_iota(jnp.int32, (rows, T), 0)
    rev = ((r & 1) << 5) | ((r & 2) << 3) | ((r & 4) << 1) \
        | ((r & 8) >> 1) | ((r & 16) >> 3) | ((r & 32) >> 5)
    base = rev << 4
    ix = [base + i for i in range(_NBLK)]

    def ce_lex(a, b):
        p = _before(v[a], ix[a], v[b], ix[b])
        hv = jnp.maximum(v[a], v[b])
        lv = jnp.minimum(v[a], v[b])
        hi = jnp.where(p, ix[a], ix[b])
        li = jnp.where(p, ix[b], ix[a])
        v[a], ix[a], v[b], ix[b] = hv, hi, lv, li

    def ce_vge(a, b):
        p = v[a] >= v[b]
        hv = jnp.maximum(v[a], v[b])
        lv = jnp.minimum(v[a], v[b])
        hi = jnp.where(p, ix[a], ix[b])
        li = jnp.where(p, ix[b], ix[a])
        v[a], ix[a], v[b], ix[b] = hv, hi, lv, li

    for (a, b), cheap in zip(_SORT16, _SORT16_CHEAP):
        (ce_vge if cheap else ce_lex)(a, b)

    half = rows // 2
    while half >= 1:
        nv, nix = [], []
        for i in range(_NBLK):
            j = _NBLK - 1 - i
            va, ia = v[i][:half], ix[i][:half]
            vb, ib = v[j][half:2 * half], ix[j][half:2 * half]
            p = va >= vb
            nv.append(jnp.maximum(va, vb))
            nix.append(jnp.where(p, ia, ib))
        v, ix = nv, nix
        for a, b in _BITONIC16:
            ce_lex(a, b)
        half //= 2

    for i in range(_K):
        vals_ref[i, :] = v[i][0]
        idx_ref[i, :] = ix[i][0]


def _topk_tc(q_raw, emb_norm):
    B, D = q_raw.shape
    V = emb_norm.shape[0]
    grid = pl.cdiv(V, _T)
    vals, idx = pl.pallas_call(
        _topk_body,
        grid=(grid,),
        in_specs=[
            pl.BlockSpec((B, D), lambda j: (0, 0)),
            pl.BlockSpec((_T, D), lambda j: (j, 0)),
        ],
        out_specs=[
            pl.BlockSpec((_K, _T), lambda j: (0, j)),
            pl.BlockSpec((_K, _T), lambda j: (0, j)),
        ],
        out_shape=[
            jax.ShapeDtypeStruct((_K, V), jnp.float32),
            jax.ShapeDtypeStruct((_K, V), jnp.int32),
        ],
    )(q_raw, emb_norm)
    return idx, vals


def kernel(token_id, emb_norm):
    tok = token_id.reshape(-1).astype(jnp.int32)
    tok = tok.reshape(_ROWS, _NBLK)[_REV6].T.reshape(-1)
    q = _gather_sc(tok, emb_norm)
    norm = jnp.linalg.norm(q, ord=2, axis=1, keepdims=True)
    qn = q / jnp.maximum(norm, 1e-12)
    idx, vals = _topk_tc(qn, emb_norm)
    return idx, vals
